# baseline (speedup 1.0000x reference)
import functools

import jax
import jax.numpy as jnp
from jax.experimental import pallas as pl
from jax.experimental.pallas import tpu as pltpu

_BS = 1024
_BB = 16
_NSEM = 8


def _tc_broadcast(table):
    num_mode, d_model = table.shape
    n_chunks = _BS // _BB
    rows = _BB * num_mode

    def body(in_ref, out_ref, hbm_stage, tab_v, stage, sem_in, sem_out):
        out2d = out_ref.reshape(_BS * num_mode, d_model)
        pltpu.make_async_copy(in_ref, tab_v, sem_in).start()
        pltpu.make_async_copy(in_ref, tab_v, sem_in).wait()
        s3 = stage.reshape(_BB, num_mode, d_model)
        s3[...] = jnp.broadcast_to(tab_v[...], (_BB, num_mode, d_model))
        pltpu.make_async_copy(stage, hbm_stage, sem_in).start()
        pltpu.make_async_copy(stage, hbm_stage, sem_in).wait()
        for i in range(n_chunks):
            pltpu.make_async_copy(
                hbm_stage, out2d.at[pl.ds(i * rows, rows)],
                sem_out.at[i % _NSEM]).start()
        for i in range(n_chunks):
            pltpu.make_async_copy(
                hbm_stage, out2d.at[pl.ds(i * rows, rows)],
                sem_out.at[i % _NSEM]).wait()

    out, _ = pl.pallas_call(
        body,
        in_specs=[pl.BlockSpec(memory_space=pltpu.HBM)],
        out_specs=[pl.BlockSpec(memory_space=pltpu.HBM),
                   pl.BlockSpec(memory_space=pltpu.HBM)],
        out_shape=[
            jax.ShapeDtypeStruct((_BS, num_mode, d_model), jnp.float32),
            jax.ShapeDtypeStruct((rows, d_model), jnp.float32),
        ],
        scratch_shapes=[
            pltpu.VMEM((num_mode, d_model), jnp.float32),
            pltpu.VMEM((rows, d_model), jnp.float32),
            pltpu.SemaphoreType.DMA,
            pltpu.SemaphoreType.DMA((_NSEM,)),
        ],
    )(table)
    return out


def kernel(mode_emb_weight, bs, num_mode):
    del bs, num_mode
    return _tc_broadcast(mode_emb_weight)


# speedup vs baseline: 29.7571x; 29.7571x over previous
import functools

import jax
import jax.numpy as jnp
from jax import lax
from jax.experimental import pallas as pl
from jax.experimental.pallas import tpu as pltpu
from jax.experimental.pallas import tpu_sc as plsc

_NC = 2
_NS = 16


def _sc_broadcast(table, bs):
    num_mode, d_model = table.shape
    nw = _NC * _NS
    b_per_w = bs // nw

    mesh = plsc.VectorSubcoreMesh(
        core_axis_name="c", subcore_axis_name="s",
        num_cores=_NC, num_subcores=_NS)

    @functools.partial(
        pl.kernel,
        out_type=jax.ShapeDtypeStruct((bs, num_mode, d_model), jnp.float32),
        mesh=mesh,
        scratch_types=[
            pltpu.VMEM((num_mode, d_model), jnp.float32),
            pltpu.SemaphoreType.DMA,
        ],
    )
    def k(table_hbm, out_hbm, tab_v, sem):
        wid = lax.axis_index("s") * _NC + lax.axis_index("c")
        base = wid * b_per_w
        pltpu.sync_copy(table_hbm, tab_v)
        copies = [
            pltpu.async_copy(tab_v, out_hbm.at[base + i], sem)
            for i in range(b_per_w)
        ]
        for c in copies:
            c.wait()

    return k(table)


_BS = 1024


def kernel(mode_emb_weight, bs, num_mode):
    del bs, num_mode
    return _sc_broadcast(mode_emb_weight, _BS)
